# baseline (speedup 1.0000x reference)
import functools

import jax
import jax.numpy as jnp
from jax import lax
from jax.experimental import pallas as pl
from jax.experimental.pallas import tpu as pltpu
from jax.experimental.pallas import tpu_sc as plsc

_IDX = 13
_NUM_CORES = 2
_NUM_SUBCORES = 16
_NUM_WORKERS = _NUM_CORES * _NUM_SUBCORES


def kernel(inputs):
    batch, fields, dim = inputs.shape
    x2d = inputs.reshape(batch, fields * dim)
    rows_per_w = batch // _NUM_WORKERS
    col = _IDX * dim

    mesh = plsc.VectorSubcoreMesh(core_axis_name="c", subcore_axis_name="s")

    @functools.partial(
        pl.kernel,
        out_type=jax.ShapeDtypeStruct((batch, dim), inputs.dtype),
        mesh=mesh,
        scratch_types=[
            pltpu.VMEM((rows_per_w, dim), inputs.dtype),
            pltpu.SemaphoreType.DMA,
        ],
        compiler_params=pltpu.CompilerParams(use_tc_tiling_on_sc=False),
    )
    def _slice_copy(x_hbm, o_hbm, buf_v, sem):
        wid = lax.axis_index("s") * _NUM_CORES + lax.axis_index("c")
        base = wid * rows_per_w
        pltpu.async_copy(
            x_hbm.at[pl.ds(base, rows_per_w), pl.ds(col, dim)], buf_v, sem
        ).wait()
        pltpu.sync_copy(buf_v, o_hbm.at[pl.ds(base, rows_per_w)])

    return _slice_copy(x2d)


# speedup vs baseline: 1.4941x; 1.4941x over previous
import functools

import jax
import jax.numpy as jnp
from jax import lax
from jax.experimental import pallas as pl
from jax.experimental.pallas import tpu as pltpu
from jax.experimental.pallas import tpu_sc as plsc

_IDX = 13
_NUM_CORES = 2
_NUM_SUBCORES = 16
_NUM_WORKERS = _NUM_CORES * _NUM_SUBCORES


def kernel(inputs):
    batch, fields, dim = inputs.shape
    x_t = jnp.transpose(inputs, (1, 2, 0))
    lanes_per_w = batch // _NUM_WORKERS

    mesh = plsc.VectorSubcoreMesh(core_axis_name="c", subcore_axis_name="s")

    @functools.partial(
        pl.kernel,
        out_type=jax.ShapeDtypeStruct((dim, batch), inputs.dtype),
        mesh=mesh,
        scratch_types=[pltpu.SemaphoreType.DMA],
    )
    def _slice_copy(x_hbm, o_hbm, sem):
        wid = lax.axis_index("s") * _NUM_CORES + lax.axis_index("c")
        base = wid * lanes_per_w
        pltpu.async_copy(
            x_hbm.at[_IDX, :, pl.ds(base, lanes_per_w)],
            o_hbm.at[:, pl.ds(base, lanes_per_w)],
            sem,
        ).wait()

    out_t = _slice_copy(x_t)
    return jnp.transpose(out_t, (1, 0))


# speedup vs baseline: 9.6086x; 6.4308x over previous
import functools

import jax
import jax.numpy as jnp
from jax import lax
from jax.experimental import pallas as pl
from jax.experimental.pallas import tpu as pltpu
from jax.experimental.pallas import tpu_sc as plsc

_IDX = 13
_NUM_CORES = 2
_NUM_SUBCORES = 16
_NUM_WORKERS = _NUM_CORES * _NUM_SUBCORES


def kernel(inputs):
    batch, fields, dim = inputs.shape
    x_t = jnp.transpose(inputs, (1, 2, 0))
    lanes_per_w = batch // _NUM_WORKERS

    mesh = plsc.VectorSubcoreMesh(core_axis_name="c", subcore_axis_name="s")

    @functools.partial(
        pl.kernel,
        out_type=jax.ShapeDtypeStruct((dim, batch), inputs.dtype),
        mesh=mesh,
        scratch_types=[
            pltpu.VMEM((dim, lanes_per_w), inputs.dtype),
            pltpu.SemaphoreType.DMA,
        ],
    )
    def _slice_copy(x_hbm, o_hbm, buf_v, sem):
        wid = lax.axis_index("s") * _NUM_CORES + lax.axis_index("c")
        base = wid * lanes_per_w
        pltpu.async_copy(
            x_hbm.at[_IDX, :, pl.ds(base, lanes_per_w)], buf_v, sem
        ).wait()
        pltpu.sync_copy(buf_v, o_hbm.at[:, pl.ds(base, lanes_per_w)])

    out_t = _slice_copy(x_t)
    return jnp.transpose(out_t, (1, 0))


# speedup vs baseline: 9.7299x; 1.0126x over previous
import functools

import jax
import jax.numpy as jnp
from jax import lax
from jax.experimental import pallas as pl
from jax.experimental.pallas import tpu as pltpu
from jax.experimental.pallas import tpu_sc as plsc

_IDX = 13
_NUM_CORES = 2
_NUM_SUBCORES = 16
_NUM_WORKERS = _NUM_CORES * _NUM_SUBCORES


def kernel(inputs):
    batch, fields, dim = inputs.shape
    x_t = jnp.transpose(inputs, (1, 2, 0))
    lanes_per_w = batch // _NUM_WORKERS

    mesh = plsc.VectorSubcoreMesh(core_axis_name="c", subcore_axis_name="s")

    sub_groups = dim // 8
    lane_groups = _NUM_WORKERS // sub_groups
    lanes_per_g = batch // lane_groups

    @functools.partial(
        pl.kernel,
        out_type=jax.ShapeDtypeStruct((dim, batch), inputs.dtype),
        mesh=mesh,
        scratch_types=[
            pltpu.VMEM((8, lanes_per_g), inputs.dtype),
            pltpu.SemaphoreType.DMA,
        ],
    )
    def _slice_copy(x_hbm, o_hbm, buf_v, sem):
        wid = lax.axis_index("s") * _NUM_CORES + lax.axis_index("c")
        tr = wid // lane_groups
        lg = wid % lane_groups
        pltpu.async_copy(
            x_hbm.at[_IDX, pl.ds(tr * 8, 8), pl.ds(lg * lanes_per_g, lanes_per_g)],
            buf_v,
            sem,
        ).wait()
        pltpu.sync_copy(
            buf_v, o_hbm.at[pl.ds(tr * 8, 8), pl.ds(lg * lanes_per_g, lanes_per_g)]
        )

    out_t = _slice_copy(x_t)
    return jnp.transpose(out_t, (1, 0))


# speedup vs baseline: 9.7495x; 1.0020x over previous
import functools

import jax
import jax.numpy as jnp
from jax import lax
from jax.experimental import pallas as pl
from jax.experimental.pallas import tpu as pltpu
from jax.experimental.pallas import tpu_sc as plsc

_IDX = 13
_NUM_CORES = 2
_NUM_SUBCORES = 16
_NUM_WORKERS = _NUM_CORES * _NUM_SUBCORES


def kernel(inputs):
    batch, fields, dim = inputs.shape
    plane = dim * batch
    chunk = plane // _NUM_WORKERS

    t = jnp.transpose(inputs, (1, 2, 0))
    t = t.reshape(fields, dim // 8, 8, batch // 128, 128)
    t = jnp.transpose(t, (0, 1, 3, 2, 4))
    flat = t.reshape(fields * plane)

    mesh = plsc.VectorSubcoreMesh(core_axis_name="c", subcore_axis_name="s")

    @functools.partial(
        pl.kernel,
        out_type=jax.ShapeDtypeStruct((plane,), inputs.dtype),
        mesh=mesh,
        scratch_types=[
            pltpu.VMEM((chunk,), inputs.dtype),
            pltpu.SemaphoreType.DMA,
        ],
        compiler_params=pltpu.CompilerParams(use_tc_tiling_on_sc=False),
    )
    def _slice_copy(x_hbm, o_hbm, buf_v, sem):
        wid = lax.axis_index("s") * _NUM_CORES + lax.axis_index("c")
        base = wid * chunk
        pltpu.async_copy(
            x_hbm.at[pl.ds(_IDX * plane + base, chunk)], buf_v, sem
        ).wait()
        pltpu.sync_copy(buf_v, o_hbm.at[pl.ds(base, chunk)])

    out_flat = _slice_copy(flat)

    o = out_flat.reshape(dim // 8, batch // 128, 8, 128)
    o = jnp.transpose(o, (0, 2, 1, 3))
    o = o.reshape(dim, batch)
    return jnp.transpose(o, (1, 0))
